# baseline (speedup 1.0000x reference)
import jax
import jax.numpy as jnp
from jax.experimental import pallas as pl
from jax.experimental.pallas import tpu as pltpu

S = 2048
RB = 128
CB = 256
N_CC = S // CB


def _body(clo_ref, chi_ref, cv_ref, x_ref, o_ref):
    rb = pl.program_id(0)
    cc = pl.program_id(1)
    skipped = (cc >= clo_ref[rb]) & (cc <= chi_ref[rb])

    @pl.when(skipped)
    def _():
        o_ref[...] = jnp.zeros_like(o_ref)

    @pl.when(jnp.logical_not(skipped))
    def _():
        cv = cv_ref[0]
        g = rb * RB + jax.lax.broadcasted_iota(jnp.int32, (RB, CB), 0)
        r = jax.lax.rem(g, S)
        i = jnp.minimum(r, S - 1 - r)
        val = jnp.clip((i.astype(jnp.float32) - 991.0 + 2048.0 * cv)
                       * (1.0 / 32.0), 0.0, 1.0)
        c = cc * CB + jax.lax.broadcasted_iota(jnp.int32, (RB, CB), 1)
        left = i + jnp.where(r >= S // 2, 1, 0)
        cond = (c >= left) & (c <= S - 1 - i)
        o_ref[...] = x_ref[...] * jnp.where(cond, val, 1.0)


def _skip_ranges(cv, n_rb):
    rb = jnp.arange(n_rb, dtype=jnp.int32)
    r0 = (rb * RB) % S
    top = (r0 + RB) <= (S // 2)
    max_i = jnp.where(top, r0 + RB - 1, S - 1 - r0)
    v = (max_i.astype(jnp.float32) - 991.0 + 2048.0 * cv[0])
    val0 = v <= 0.0
    max_left = max_i + jnp.where(top, 0, 1)
    min_right = S - 1 - max_i
    clo = (max_left + CB - 1) // CB
    chi = (min_right - (CB - 1)) // CB
    ok = val0 & (clo <= chi)
    clo = jnp.where(ok, clo, 1).astype(jnp.int32)
    chi = jnp.where(ok, chi, 0).astype(jnp.int32)
    return clo, chi


def kernel(x, current_val):
    B, H, Sr, Sc = x.shape
    x2 = x.reshape(B * H * Sr, Sc)
    n_rows = x2.shape[0]
    n_rb = n_rows // RB
    clo, chi = _skip_ranges(current_val, n_rb)

    def x_map(rb, cc, clo_ref, chi_ref):
        skipped = (cc >= clo_ref[rb]) & (cc <= chi_ref[rb])
        return rb, jnp.where(skipped, clo_ref[rb] - 1, cc)

    def o_map(rb, cc, clo_ref, chi_ref):
        return rb, cc

    grid_spec = pltpu.PrefetchScalarGridSpec(
        num_scalar_prefetch=2,
        grid=(n_rb, N_CC),
        in_specs=[
            pl.BlockSpec(memory_space=pltpu.SMEM),
            pl.BlockSpec((RB, CB), x_map),
        ],
        out_specs=pl.BlockSpec((RB, CB), o_map),
    )
    out = pl.pallas_call(
        _body,
        grid_spec=grid_spec,
        out_shape=jax.ShapeDtypeStruct((n_rows, Sc), x.dtype),
    )(clo, chi, current_val, x2)
    return out.reshape(B, H, Sr, Sc)
